# Initial kernel scaffold; baseline (speedup 1.0000x reference)
import jax
import jax.numpy as jnp
from jax.experimental import pallas as pl


def kernel(x, m, table, W, b):
    raise NotImplementedError("write your pallas kernel here")


# speedup vs baseline: 2.2845x; 2.2845x over previous
import functools

import jax
import jax.numpy as jnp
from jax import lax
from jax.experimental import pallas as pl
from jax.experimental.pallas import tpu as pltpu
from jax.experimental.pallas import tpu_sc as plsc

_B = 4096
_L = 200
_D = 32
_NL = 10
_NW = 32
_BPW = _B // _NW
_C0 = 128
_C1 = _L - _C0
_UNROLL = 8

_mesh = plsc.VectorSubcoreMesh(core_axis_name="c", subcore_axis_name="s")


@functools.partial(
    pl.kernel,
    out_type=jax.ShapeDtypeStruct((_B, _D), jnp.float32),
    mesh=_mesh,
    scratch_types=[
        pltpu.VMEM((_BPW, _L), jnp.int32),
        pltpu.VMEM((2, _L, _D), jnp.float32),
        pltpu.VMEM((_BPW, _D), jnp.float32),
        pltpu.SemaphoreType.DMA,
        pltpu.SemaphoreType.DMA,
    ],
    compiler_params=pltpu.CompilerParams(use_tc_tiling_on_sc=False),
)
def _embed_sum(x_hbm, table_hbm, doc_hbm, idx_v, rows_v, doc_v, sem0, sem1):
    wid = lax.axis_index("s") * 2 + lax.axis_index("c")
    base = wid * _BPW
    pltpu.sync_copy(x_hbm.at[pl.ds(base, _BPW)], idx_v)

    sems = (sem0, sem1)

    def descs(r, p, sem):
        d0 = pltpu.make_async_copy(
            table_hbm.at[idx_v.at[r, pl.ds(0, _C0)]],
            rows_v.at[p, pl.ds(0, _C0)], sem)
        d1 = pltpu.make_async_copy(
            table_hbm.at[idx_v.at[r, pl.ds(_C0, _C1)]],
            rows_v.at[p, pl.ds(_C0, _C1)], sem)
        return d0, d1

    def issue(r, p, sem):
        d0, d1 = descs(r, p, sem)
        d0.start()
        d1.start()

    issue(0, 0, sem0)
    issue(1, 1, sem1)

    zeros = jnp.zeros((16,), jnp.float32)

    def outer(g, carry):
        for p in range(2):
            r = g * 2 + p
            d0, d1 = descs(r, p, sems[p])
            d0.wait()
            d1.wait()

            def rbody(jj, acc):
                a0, a1, b0, b1 = acc
                for u in range(_UNROLL):
                    j = jj * _UNROLL + u
                    lo = rows_v[p, j, pl.ds(0, 16)]
                    hi = rows_v[p, j, pl.ds(16, 16)]
                    if u % 2 == 0:
                        a0 = a0 + lo
                        a1 = a1 + hi
                    else:
                        b0 = b0 + lo
                        b1 = b1 + hi
                return (a0, a1, b0, b1)

            a0, a1, b0, b1 = lax.fori_loop(
                0, _L // _UNROLL, rbody, (zeros, zeros, zeros, zeros))

            @pl.when(r + 2 < _BPW)
            def _():
                issue(r + 2, p, sems[p])

            doc_v[r, pl.ds(0, 16)] = a0 + b0
            doc_v[r, pl.ds(16, 16)] = a1 + b1
        return carry

    lax.fori_loop(0, _BPW // 2, outer, 0)
    pltpu.sync_copy(doc_v, doc_hbm.at[pl.ds(base, _BPW)])


def _head_body(doc_ref, w_ref, b_ref, out_ref):
    out_ref[...] = (
        jnp.dot(doc_ref[...], w_ref[...], preferred_element_type=jnp.float32)
        + b_ref[...]
    )


def _head(doc, W, b):
    nblk = 4
    return pl.pallas_call(
        _head_body,
        out_shape=jax.ShapeDtypeStruct((_B, _NL), jnp.float32),
        grid=(nblk,),
        in_specs=[
            pl.BlockSpec((_B // nblk, _D), lambda i: (i, 0)),
            pl.BlockSpec((_D, _NL), lambda i: (0, 0)),
            pl.BlockSpec((1, _NL), lambda i: (0, 0)),
        ],
        out_specs=pl.BlockSpec((_B // nblk, _NL), lambda i: (i, 0)),
    )(doc, W, b.reshape(1, _NL))


def kernel(x, m, table, W, b):
    del m
    x = x.astype(jnp.int32)
    doc = _embed_sum(x, table)
    return _head(doc, W, b)
